# baseline (speedup 1.0000x reference)
import functools

import jax
import jax.numpy as jnp
from jax.experimental import pallas as pl
from jax.experimental.pallas import tpu as pltpu

N_EMBD = 768
N_HEAD = 12
N_KV_HEAD = 4
FFN = 128
N_EXP = 64
TOP_K = 2
HEAD_DIM = N_EMBD // N_HEAD
KV_DIM = N_KV_HEAD * HEAD_DIM
N_GROUPS = N_HEAD // N_KV_HEAD
KERNEL = 3
BASE = 50000.0

QT = 128


def _rms_norm(x, w):
    return x * jax.lax.rsqrt(jnp.mean(x * x, axis=-1, keepdims=True) + 1e-6) * w


def _ln_na(x, eps=1e-5):
    m = jnp.mean(x, axis=-1, keepdims=True)
    v = jnp.mean((x - m) ** 2, axis=-1, keepdims=True)
    return (x - m) / jnp.sqrt(v + eps)


def _rope(x, cos, sin):
    h = x.shape[-1] // 2
    x1 = x[:, :h]
    x2 = x[:, h:]
    rot = jnp.concatenate((-x2, x1), axis=-1)
    return x * cos + rot * sin


def _prenorm_conv_body(x_ref, ln1_ref, w_ref, out_ref):
    x = x_ref[...]
    xn = _rms_norm(x, ln1_ref[...])
    T = x.shape[0]
    w0 = w_ref[0:1, :]
    w1 = w_ref[1:2, :]
    w2 = w_ref[2:3, :]
    z1 = jnp.concatenate((jnp.zeros((1, xn.shape[1]), xn.dtype), xn[: T - 1]), 0)
    z2 = jnp.concatenate((jnp.zeros((2, xn.shape[1]), xn.dtype), xn[: T - 2]), 0)
    out_ref[...] = xn * w2 + z1 * w1 + z2 * w0


def _qkv_body(xc_ref, w_ref, cos_ref, sin_ref, q_ref, k_ref, v_ref):
    qkv = jnp.dot(xc_ref[...], w_ref[...].T, preferred_element_type=jnp.float32)
    cos = cos_ref[...]
    sin = sin_ref[...]
    for h in range(N_HEAD):
        qh = qkv[:, h * HEAD_DIM:(h + 1) * HEAD_DIM]
        q_ref[h] = _ln_na(_rope(qh, cos, sin))
    for h in range(N_KV_HEAD):
        kh = qkv[:, N_EMBD + h * HEAD_DIM:N_EMBD + (h + 1) * HEAD_DIM]
        k_ref[h] = _ln_na(_rope(kh, cos, sin))
        v_ref[h] = qkv[:, N_EMBD + KV_DIM + h * HEAD_DIM:N_EMBD + KV_DIM + (h + 1) * HEAD_DIM]


def _attn_body(q_ref, k_ref, v_ref, o_ref):
    qt = pl.program_id(1)
    q = q_ref[0]
    k = k_ref[0]
    v = v_ref[0]
    s = jax.lax.dot_general(q, k, (((1,), (1,)), ((), ())),
                            preferred_element_type=jnp.float32)
    s = s * (1.0 / jnp.sqrt(jnp.float32(HEAD_DIM)))
    rows = qt * QT + jax.lax.broadcasted_iota(jnp.int32, s.shape, 0)
    cols = jax.lax.broadcasted_iota(jnp.int32, s.shape, 1)
    s = jnp.where(cols <= rows, s, -jnp.inf)
    m = jnp.max(s, axis=-1, keepdims=True)
    e = jnp.exp(s - m)
    p = e / jnp.sum(e, axis=-1, keepdims=True)
    o_ref[0] = jnp.dot(p, v, preferred_element_type=jnp.float32)


def _proj_router_body(y_ref, x_ref, wp_ref, ln2_ref, wr_ref,
                      x1_ref, xn2_ref, gate_ref, aux_ref):
    y = jnp.dot(y_ref[...], wp_ref[...].T, preferred_element_type=jnp.float32)
    x1 = x_ref[...] + y
    x1_ref[...] = x1
    xn2 = _rms_norm(x1, ln2_ref[...])
    xn2_ref[...] = xn2
    logits = jnp.dot(xn2, wr_ref[...].T, preferred_element_type=jnp.float32)
    ml = jnp.max(logits, axis=-1, keepdims=True)
    el = jnp.exp(logits - ml)
    probs = el / jnp.sum(el, axis=-1, keepdims=True)
    col = jax.lax.broadcasted_iota(jnp.int32, probs.shape, 1)
    m1 = jnp.max(probs, axis=-1, keepdims=True)
    i1 = jnp.min(jnp.where(probs == m1, col, N_EXP), axis=-1, keepdims=True)
    mask1 = col == i1
    p2 = jnp.where(mask1, -1.0, probs)
    m2 = jnp.max(p2, axis=-1, keepdims=True)
    i2 = jnp.min(jnp.where(p2 == m2, col, N_EXP), axis=-1, keepdims=True)
    mask2 = col == i2
    denom = m1 + m2
    gate_ref[...] = jnp.where(mask1, m1 / denom, 0.0) + jnp.where(mask2, m2 / denom, 0.0)
    avg = jnp.mean(probs, axis=0, keepdims=True)
    aux_ref[...] = N_EXP * jnp.sum(avg * avg, axis=-1, keepdims=True)


def _moe_dense_body(xn2_ref, gate_ref, wv_ref, wp_ref, out_ref):
    e = pl.program_id(0)
    xn2 = xn2_ref[...]
    h = jax.lax.dot_general(xn2, wv_ref[0], (((1,), (1,)), ((), ())),
                            preferred_element_type=jnp.float32)
    g = h[:, :FFN]
    val = h[:, FFN:]
    act = g * jax.lax.logistic(g) * val
    gate = gate_ref[...]
    ecol = jax.lax.broadcasted_iota(jnp.int32, gate.shape, 1)
    gcol = jnp.sum(jnp.where(ecol == e, gate, 0.0), axis=-1, keepdims=True)
    act = act * gcol
    contrib = jax.lax.dot_general(act, wp_ref[0], (((1,), (1,)), ((), ())),
                                  preferred_element_type=jnp.float32)

    @pl.when(e == 0)
    def _():
        out_ref[...] = contrib

    @pl.when(e != 0)
    def _():
        out_ref[...] += contrib


def _rope_tables(T):
    inv_freq = 1.0 / (BASE ** (jnp.arange(0, HEAD_DIM, 2, dtype=jnp.float32) / HEAD_DIM))
    t = jnp.arange(T, dtype=jnp.float32)
    freqs = jnp.outer(t, inv_freq)
    emb = jnp.concatenate((freqs, freqs), axis=-1)
    return jnp.cos(emb), jnp.sin(emb)


def kernel(x, ln1_w, ln2_w, l_conv_w, c_attn_w, c_proj_w, router_w, expert_wv, expert_wproj):
    B, T, C = x.shape
    x2d = x.reshape(T, C)
    ln1 = ln1_w.reshape(1, C)
    ln2 = ln2_w.reshape(1, C)
    wconv = jnp.transpose(l_conv_w[:, 0, :], (1, 0))
    cos, sin = _rope_tables(T)

    xc = pl.pallas_call(
        _prenorm_conv_body,
        out_shape=jax.ShapeDtypeStruct((T, C), jnp.float32),
    )(x2d, ln1, wconv)

    MT = 128
    q, k, v = pl.pallas_call(
        _qkv_body,
        grid=(T // MT,),
        in_specs=[
            pl.BlockSpec((MT, C), lambda i: (i, 0)),
            pl.BlockSpec((N_EMBD + 2 * KV_DIM, C), lambda i: (0, 0)),
            pl.BlockSpec((MT, HEAD_DIM), lambda i: (i, 0)),
            pl.BlockSpec((MT, HEAD_DIM), lambda i: (i, 0)),
        ],
        out_specs=[
            pl.BlockSpec((N_HEAD, MT, HEAD_DIM), lambda i: (0, i, 0)),
            pl.BlockSpec((N_KV_HEAD, MT, HEAD_DIM), lambda i: (0, i, 0)),
            pl.BlockSpec((N_KV_HEAD, MT, HEAD_DIM), lambda i: (0, i, 0)),
        ],
        out_shape=[
            jax.ShapeDtypeStruct((N_HEAD, T, HEAD_DIM), jnp.float32),
            jax.ShapeDtypeStruct((N_KV_HEAD, T, HEAD_DIM), jnp.float32),
            jax.ShapeDtypeStruct((N_KV_HEAD, T, HEAD_DIM), jnp.float32),
        ],
    )(xc, c_attn_w, cos, sin)

    y3 = pl.pallas_call(
        _attn_body,
        grid=(N_HEAD, T // QT),
        in_specs=[
            pl.BlockSpec((1, QT, HEAD_DIM), lambda h, i: (h, i, 0)),
            pl.BlockSpec((1, T, HEAD_DIM), lambda h, i: (h // N_GROUPS, 0, 0)),
            pl.BlockSpec((1, T, HEAD_DIM), lambda h, i: (h // N_GROUPS, 0, 0)),
        ],
        out_specs=pl.BlockSpec((1, QT, HEAD_DIM), lambda h, i: (h, i, 0)),
        out_shape=jax.ShapeDtypeStruct((N_HEAD, T, HEAD_DIM), jnp.float32),
    )(q, k, v)
    y = jnp.transpose(y3, (1, 0, 2)).reshape(T, N_EMBD)

    x1, xn2, gate, aux = pl.pallas_call(
        _proj_router_body,
        out_shape=[
            jax.ShapeDtypeStruct((T, C), jnp.float32),
            jax.ShapeDtypeStruct((T, C), jnp.float32),
            jax.ShapeDtypeStruct((T, N_EXP), jnp.float32),
            jax.ShapeDtypeStruct((1, 1), jnp.float32),
        ],
    )(y, x2d, c_proj_w, ln2, router_w)

    moe_out = pl.pallas_call(
        _moe_dense_body,
        grid=(N_EXP,),
        in_specs=[
            pl.BlockSpec((T, C), lambda e: (0, 0)),
            pl.BlockSpec((T, N_EXP), lambda e: (0, 0)),
            pl.BlockSpec((1, 2 * FFN, C), lambda e: (e, 0, 0)),
            pl.BlockSpec((1, C, FFN), lambda e: (e, 0, 0)),
        ],
        out_specs=pl.BlockSpec((T, C), lambda e: (0, 0)),
        out_shape=jax.ShapeDtypeStruct((T, C), jnp.float32),
    )(xn2, gate, expert_wv, expert_wproj)

    out = (x1 + moe_out).reshape(B, T, C)
    return out, aux[0, 0]


# speedup vs baseline: 1.2447x; 1.2447x over previous
import functools

import jax
import jax.numpy as jnp
from jax import lax
from jax.experimental import pallas as pl
from jax.experimental.pallas import tpu as pltpu
from jax.experimental.pallas import tpu_sc as plsc

N_EMBD = 768
N_HEAD = 12
N_KV_HEAD = 4
FFN = 128
N_EXP = 64
TOP_K = 2
HEAD_DIM = N_EMBD // N_HEAD
KV_DIM = N_KV_HEAD * HEAD_DIM
N_GROUPS = N_HEAD // N_KV_HEAD
KERNEL = 3
BASE = 50000.0

QT = 128
TILE = 128
SC_NC = 2
SC_NS = 16


def _rms_norm(x, w):
    return x * jax.lax.rsqrt(jnp.mean(x * x, axis=-1, keepdims=True) + 1e-6) * w


def _ln_na(x, eps=1e-5):
    m = jnp.mean(x, axis=-1, keepdims=True)
    v = jnp.mean((x - m) ** 2, axis=-1, keepdims=True)
    return (x - m) / jnp.sqrt(v + eps)


def _rope(x, cos, sin):
    h = x.shape[-1] // 2
    x1 = x[:, :h]
    x2 = x[:, h:]
    rot = jnp.concatenate((-x2, x1), axis=-1)
    return x * cos + rot * sin


def _prenorm_conv_body(x_ref, ln1_ref, w_ref, out_ref):
    x = x_ref[...]
    xn = _rms_norm(x, ln1_ref[...])
    T = x.shape[0]
    w0 = w_ref[0:1, :]
    w1 = w_ref[1:2, :]
    w2 = w_ref[2:3, :]
    z1 = jnp.concatenate((jnp.zeros((1, xn.shape[1]), xn.dtype), xn[: T - 1]), 0)
    z2 = jnp.concatenate((jnp.zeros((2, xn.shape[1]), xn.dtype), xn[: T - 2]), 0)
    out_ref[...] = xn * w2 + z1 * w1 + z2 * w0


def _qkv_body(xc_ref, w_ref, cos_ref, sin_ref, q_ref, k_ref, v_ref):
    qkv = jnp.dot(xc_ref[...], w_ref[...].T, preferred_element_type=jnp.float32)
    cos = cos_ref[...]
    sin = sin_ref[...]
    for h in range(N_HEAD):
        qh = qkv[:, h * HEAD_DIM:(h + 1) * HEAD_DIM]
        q_ref[h] = _ln_na(_rope(qh, cos, sin))
    for h in range(N_KV_HEAD):
        kh = qkv[:, N_EMBD + h * HEAD_DIM:N_EMBD + (h + 1) * HEAD_DIM]
        k_ref[h] = _ln_na(_rope(kh, cos, sin))
        v_ref[h] = qkv[:, N_EMBD + KV_DIM + h * HEAD_DIM:N_EMBD + KV_DIM + (h + 1) * HEAD_DIM]


def _attn_body(q_ref, k_ref, v_ref, o_ref):
    qt = pl.program_id(1)
    q = q_ref[0]
    k = k_ref[0]
    v = v_ref[0]
    s = jax.lax.dot_general(q, k, (((1,), (1,)), ((), ())),
                            preferred_element_type=jnp.float32)
    s = s * (1.0 / jnp.sqrt(jnp.float32(HEAD_DIM)))
    rows = qt * QT + jax.lax.broadcasted_iota(jnp.int32, s.shape, 0)
    cols = jax.lax.broadcasted_iota(jnp.int32, s.shape, 1)
    s = jnp.where(cols <= rows, s, -jnp.inf)
    m = jnp.max(s, axis=-1, keepdims=True)
    e = jnp.exp(s - m)
    p = e / jnp.sum(e, axis=-1, keepdims=True)
    o_ref[0] = jnp.dot(p, v, preferred_element_type=jnp.float32)


def _proj_router_body(y_ref, x_ref, wp_ref, ln2_ref, wr_ref,
                      x1_ref, xn2_ref, p0_ref, p1_ref, w0_ref, w1_ref,
                      cnt_ref, off_ref, aux_ref, cb_ref):
    T = y_ref.shape[0]
    y = jnp.dot(y_ref[...], wp_ref[...].T, preferred_element_type=jnp.float32)
    x1 = x_ref[...] + y
    x1_ref[...] = x1
    xn2 = _rms_norm(x1, ln2_ref[...])
    xn2_ref[...] = xn2
    logits = jnp.dot(xn2, wr_ref[...].T, preferred_element_type=jnp.float32)
    ml = jnp.max(logits, axis=-1, keepdims=True)
    el = jnp.exp(logits - ml)
    probs = el / jnp.sum(el, axis=-1, keepdims=True)
    col = jax.lax.broadcasted_iota(jnp.int32, probs.shape, 1)
    m1 = jnp.max(probs, axis=-1, keepdims=True)
    i1 = jnp.min(jnp.where(probs == m1, col, N_EXP), axis=-1, keepdims=True)
    mask1 = col == i1
    pm = jnp.where(mask1, -1.0, probs)
    m2 = jnp.max(pm, axis=-1, keepdims=True)
    i2 = jnp.min(jnp.where(pm == m2, col, N_EXP), axis=-1, keepdims=True)
    mask2 = col == i2
    denom = m1 + m2
    w0_ref[...] = m1 / denom
    w1_ref[...] = m2 / denom
    avg = jnp.mean(probs, axis=0, keepdims=True)
    aux_ref[...] = N_EXP * jnp.sum(avg * avg, axis=-1, keepdims=True)

    oh = mask1.astype(jnp.float32) + mask2.astype(jnp.float32)
    r_i = jax.lax.broadcasted_iota(jnp.int32, (TILE, TILE), 0)
    c_i = jax.lax.broadcasted_iota(jnp.int32, (TILE, TILE), 1)
    ls = (r_i > c_i).astype(jnp.float32)
    carry = jnp.zeros((1, N_EXP), jnp.float32)
    for c in range(T // TILE):
        blk = oh[c * TILE:(c + 1) * TILE]
        cb_ref[c * TILE:(c + 1) * TILE, :] = carry + jnp.dot(
            ls, blk, preferred_element_type=jnp.float32)
        carry = carry + jnp.sum(blk, axis=0, keepdims=True)
    cnt = carry
    e_r = jax.lax.broadcasted_iota(jnp.int32, (N_EXP, N_EXP), 0)
    e_c = jax.lax.broadcasted_iota(jnp.int32, (N_EXP, N_EXP), 1)
    ue = (e_r < e_c).astype(jnp.float32)
    off = jnp.dot(cnt, ue, preferred_element_type=jnp.float32)
    base = off + cb_ref[...]
    p0 = jnp.sum(jnp.where(mask1, base, 0.0), axis=-1, keepdims=True)
    p1 = jnp.sum(jnp.where(mask2, base, 0.0), axis=-1, keepdims=True)
    p0_ref[...] = p0.astype(jnp.int32)
    p1_ref[...] = p1.astype(jnp.int32)
    cnt_ref[...] = cnt.astype(jnp.int32)
    off_ref[...] = off.astype(jnp.int32)


def _gmm_body(meta_ref, xg_ref, wv_ref, wp_ref, yg_ref):
    g = pl.program_id(0)
    mlo = meta_ref[2, g]
    mhi = meta_ref[3, g]
    first = meta_ref[4, g]
    rbase = meta_ref[1, g] * TILE
    xg = xg_ref[...]
    h = jax.lax.dot_general(xg, wv_ref[0], (((1,), (1,)), ((), ())),
                            preferred_element_type=jnp.float32)
    gp = h[:, :FFN]
    vp = h[:, FFN:]
    act = gp * jax.lax.logistic(gp) * vp
    rows = rbase + jax.lax.broadcasted_iota(jnp.int32, (TILE, 1), 0)
    act = jnp.where((rows >= mlo) & (rows < mhi), act, 0.0)
    contrib = jax.lax.dot_general(act, wp_ref[0], (((1,), (1,)), ((), ())),
                                  preferred_element_type=jnp.float32)

    @pl.when(first == 1)
    def _():
        yg_ref[...] = contrib

    @pl.when(first == 0)
    def _():
        yg_ref[...] += contrib


def _combine_body(x1_ref, y0_ref, y1_ref, w0_ref, w1_ref, out_ref):
    out_ref[...] = (x1_ref[...] + w0_ref[...] * y0_ref[...]
                    + w1_ref[...] * y1_ref[...])


def _sc_mesh():
    return plsc.VectorSubcoreMesh(core_axis_name="c", subcore_axis_name="s")


def _sc_dispatch(xn2, p0, p1):
    T, C = xn2.shape
    NW = SC_NC * SC_NS
    CH = T // NW

    @functools.partial(
        pl.kernel, mesh=_sc_mesh(),
        out_type=jax.ShapeDtypeStruct((TOP_K * T, C), jnp.float32),
        scratch_types=[
            pltpu.VMEM((CH,), jnp.int32),
            pltpu.VMEM((CH, C), jnp.float32),
            pltpu.SemaphoreType.DMA,
        ],
    )
    def k(xn2_hbm, p0_hbm, p1_hbm, xg_hbm, idx_v, rows_v, sem):
        wid = lax.axis_index("s") * SC_NC + lax.axis_index("c")
        rb = wid * CH
        pltpu.sync_copy(xn2_hbm.at[pl.ds(rb, CH)], rows_v)
        pltpu.sync_copy(p0_hbm.at[pl.ds(rb, CH)], idx_v)
        pltpu.async_copy(rows_v, xg_hbm.at[idx_v], sem).wait()
        pltpu.sync_copy(p1_hbm.at[pl.ds(rb, CH)], idx_v)
        pltpu.async_copy(rows_v, xg_hbm.at[idx_v], sem).wait()

    return k(xn2, p0, p1)


def _sc_combine(yg, p0, p1):
    A, C = yg.shape
    T = A // TOP_K
    NW = SC_NC * SC_NS
    CH = T // NW

    @functools.partial(
        pl.kernel, mesh=_sc_mesh(),
        out_type=[
            jax.ShapeDtypeStruct((T, C), jnp.float32),
            jax.ShapeDtypeStruct((T, C), jnp.float32),
        ],
        scratch_types=[
            pltpu.VMEM((CH,), jnp.int32),
            pltpu.VMEM((CH, C), jnp.float32),
            pltpu.SemaphoreType.DMA,
        ],
    )
    def k(yg_hbm, p0_hbm, p1_hbm, y0_hbm, y1_hbm, idx_v, buf_v, sem):
        wid = lax.axis_index("s") * SC_NC + lax.axis_index("c")
        rb = wid * CH
        pltpu.sync_copy(p0_hbm.at[pl.ds(rb, CH)], idx_v)
        pltpu.async_copy(yg_hbm.at[idx_v], buf_v, sem).wait()
        pltpu.sync_copy(buf_v, y0_hbm.at[pl.ds(rb, CH)])
        pltpu.sync_copy(p1_hbm.at[pl.ds(rb, CH)], idx_v)
        pltpu.async_copy(yg_hbm.at[idx_v], buf_v, sem).wait()
        pltpu.sync_copy(buf_v, y1_hbm.at[pl.ds(rb, CH)])

    return k(yg, p0, p1)


def _pair_metadata(cnt, off, n_tiles):
    G = n_tiles + N_EXP
    end = off + cnt
    tlo = off // TILE
    thi = jnp.where(cnt > 0, (end - 1) // TILE, 0)
    p = jnp.where(cnt > 0, thi - tlo + 1, 0)
    s = jnp.cumsum(p) - p
    j = jnp.arange(G, dtype=jnp.int32)[:, None]
    sel = (j >= s[None, :]) & (j < (s + p)[None, :])
    valid = jnp.any(sel, axis=1)
    pair_e = jnp.argmax(sel, axis=1).astype(jnp.int32)
    pair_m = jnp.where(valid, tlo[pair_e] + (j[:, 0] - s[pair_e]), n_tiles - 1)
    mlo = jnp.where(valid, off[pair_e], 0)
    mhi = jnp.where(valid, end[pair_e], 0)
    first = jnp.concatenate(
        [jnp.ones((1,), jnp.int32),
         (pair_m[1:] != pair_m[:-1]).astype(jnp.int32)])
    return jnp.stack([pair_e, pair_m.astype(jnp.int32),
                      mlo.astype(jnp.int32), mhi.astype(jnp.int32), first])


def _rope_tables(T):
    inv_freq = 1.0 / (BASE ** (jnp.arange(0, HEAD_DIM, 2, dtype=jnp.float32) / HEAD_DIM))
    t = jnp.arange(T, dtype=jnp.float32)
    freqs = jnp.outer(t, inv_freq)
    emb = jnp.concatenate((freqs, freqs), axis=-1)
    return jnp.cos(emb), jnp.sin(emb)


def kernel(x, ln1_w, ln2_w, l_conv_w, c_attn_w, c_proj_w, router_w, expert_wv, expert_wproj):
    B, T, C = x.shape
    x2d = x.reshape(T, C)
    ln1 = ln1_w.reshape(1, C)
    ln2 = ln2_w.reshape(1, C)
    wconv = jnp.transpose(l_conv_w[:, 0, :], (1, 0))
    cos, sin = _rope_tables(T)

    xc = pl.pallas_call(
        _prenorm_conv_body,
        out_shape=jax.ShapeDtypeStruct((T, C), jnp.float32),
    )(x2d, ln1, wconv)

    MT = 128
    q, k, v = pl.pallas_call(
        _qkv_body,
        grid=(T // MT,),
        in_specs=[
            pl.BlockSpec((MT, C), lambda i: (i, 0)),
            pl.BlockSpec((N_EMBD + 2 * KV_DIM, C), lambda i: (0, 0)),
            pl.BlockSpec((MT, HEAD_DIM), lambda i: (i, 0)),
            pl.BlockSpec((MT, HEAD_DIM), lambda i: (i, 0)),
        ],
        out_specs=[
            pl.BlockSpec((N_HEAD, MT, HEAD_DIM), lambda i: (0, i, 0)),
            pl.BlockSpec((N_KV_HEAD, MT, HEAD_DIM), lambda i: (0, i, 0)),
            pl.BlockSpec((N_KV_HEAD, MT, HEAD_DIM), lambda i: (0, i, 0)),
        ],
        out_shape=[
            jax.ShapeDtypeStruct((N_HEAD, T, HEAD_DIM), jnp.float32),
            jax.ShapeDtypeStruct((N_KV_HEAD, T, HEAD_DIM), jnp.float32),
            jax.ShapeDtypeStruct((N_KV_HEAD, T, HEAD_DIM), jnp.float32),
        ],
    )(xc, c_attn_w, cos, sin)

    y3 = pl.pallas_call(
        _attn_body,
        grid=(N_HEAD, T // QT),
        in_specs=[
            pl.BlockSpec((1, QT, HEAD_DIM), lambda h, i: (h, i, 0)),
            pl.BlockSpec((1, T, HEAD_DIM), lambda h, i: (h // N_GROUPS, 0, 0)),
            pl.BlockSpec((1, T, HEAD_DIM), lambda h, i: (h // N_GROUPS, 0, 0)),
        ],
        out_specs=pl.BlockSpec((1, QT, HEAD_DIM), lambda h, i: (h, i, 0)),
        out_shape=jax.ShapeDtypeStruct((N_HEAD, T, HEAD_DIM), jnp.float32),
    )(q, k, v)
    y = jnp.transpose(y3, (1, 0, 2)).reshape(T, N_EMBD)

    x1, xn2, p0, p1, w0, w1, cnt, off, aux = pl.pallas_call(
        _proj_router_body,
        out_shape=[
            jax.ShapeDtypeStruct((T, C), jnp.float32),
            jax.ShapeDtypeStruct((T, C), jnp.float32),
            jax.ShapeDtypeStruct((T, 1), jnp.int32),
            jax.ShapeDtypeStruct((T, 1), jnp.int32),
            jax.ShapeDtypeStruct((T, 1), jnp.float32),
            jax.ShapeDtypeStruct((T, 1), jnp.float32),
            jax.ShapeDtypeStruct((1, N_EXP), jnp.int32),
            jax.ShapeDtypeStruct((1, N_EXP), jnp.int32),
            jax.ShapeDtypeStruct((1, 1), jnp.float32),
        ],
        scratch_shapes=[pltpu.VMEM((T, N_EXP), jnp.float32)],
    )(y, x2d, c_proj_w, ln2, router_w)

    A = TOP_K * T
    n_tiles = A // TILE
    meta = _pair_metadata(cnt[0], off[0], n_tiles)

    p0f = p0.reshape(T)
    p1f = p1.reshape(T)
    xg = _sc_dispatch(xn2, p0f, p1f)

    G = n_tiles + N_EXP
    yg = pl.pallas_call(
        _gmm_body,
        grid_spec=pltpu.PrefetchScalarGridSpec(
            num_scalar_prefetch=1,
            grid=(G,),
            in_specs=[
                pl.BlockSpec((TILE, C), lambda g, m: (m[1, g], 0)),
                pl.BlockSpec((1, 2 * FFN, C), lambda g, m: (m[0, g], 0, 0)),
                pl.BlockSpec((1, C, FFN), lambda g, m: (m[0, g], 0, 0)),
            ],
            out_specs=pl.BlockSpec((TILE, C), lambda g, m: (m[1, g], 0)),
        ),
        out_shape=jax.ShapeDtypeStruct((A, C), jnp.float32),
    )(meta, xg, expert_wv, expert_wproj)

    y0, y1 = _sc_combine(yg, p0f, p1f)

    out2d = pl.pallas_call(
        _combine_body,
        grid=(T // 256,),
        in_specs=[
            pl.BlockSpec((256, C), lambda i: (i, 0)),
            pl.BlockSpec((256, C), lambda i: (i, 0)),
            pl.BlockSpec((256, C), lambda i: (i, 0)),
            pl.BlockSpec((256, 1), lambda i: (i, 0)),
            pl.BlockSpec((256, 1), lambda i: (i, 0)),
        ],
        out_specs=pl.BlockSpec((256, C), lambda i: (i, 0)),
        out_shape=jax.ShapeDtypeStruct((T, C), jnp.float32),
    )(x1, y0, y1, w0, w1)

    return out2d.reshape(B, T, C), aux[0, 0]
